# baseline (device time: 1553563 ns/iter reference)
import jax
import jax.numpy as jnp
from jax import lax
from jax.experimental import pallas as pl
from jax.experimental.pallas import tpu as pltpu

N_DEV = 4


def _neighbor_barrier(left, right):
    barrier_sem = pltpu.get_barrier_semaphore()
    for nbr in [left, right]:
        pl.semaphore_signal(
            barrier_sem, inc=1,
            device_id=(nbr,), device_id_type=pl.DeviceIdType.MESH,
        )
    pl.semaphore_wait(barrier_sem, 2)


def _rdma(src, dst, send_sem, recv_sem, dev):
    return pltpu.make_async_remote_copy(
        src_ref=src, dst_ref=dst, send_sem=send_sem, recv_sem=recv_sem,
        device_id=(dev,), device_id_type=pl.DeviceIdType.MESH,
    )


def _ag_w_body(w_ref, wg_ref, local_sem, send_sems, recv_sems):
    my = lax.axis_index("i")
    left = (my - 1) % N_DEV
    right = (my + 1) % N_DEV
    opp = (my + 2) % N_DEV
    k, n_per = w_ref.shape
    half = k // 2
    top = pl.ds(0, half)
    bot = pl.ds(half, half)

    _neighbor_barrier(left, right)

    cp = pltpu.make_async_copy(w_ref, wg_ref.at[my], local_sem)
    cp.start()

    s_r1 = _rdma(w_ref, wg_ref.at[my], send_sems.at[0], recv_sems.at[0], right)
    s_l1 = _rdma(w_ref, wg_ref.at[my], send_sems.at[1], recv_sems.at[1], left)
    s_r1.start()
    s_l1.start()

    r_l1 = _rdma(wg_ref.at[left], wg_ref.at[left],
                 send_sems.at[0], recv_sems.at[0], left)
    r_l1.wait_recv()
    s_r2 = _rdma(wg_ref.at[left, top], wg_ref.at[left, top],
                 send_sems.at[2], recv_sems.at[2], right)
    s_r2.start()

    r_r1 = _rdma(wg_ref.at[right], wg_ref.at[right],
                 send_sems.at[1], recv_sems.at[1], right)
    r_r1.wait_recv()
    s_l2 = _rdma(wg_ref.at[right, bot], wg_ref.at[right, bot],
                 send_sems.at[3], recv_sems.at[3], left)
    s_l2.start()

    r_l2 = _rdma(wg_ref.at[opp, top], wg_ref.at[opp, top],
                 send_sems.at[2], recv_sems.at[2], left)
    r_l2.wait_recv()
    r_r2 = _rdma(wg_ref.at[opp, bot], wg_ref.at[opp, bot],
                 send_sems.at[3], recv_sems.at[3], right)
    r_r2.wait_recv()

    for s in (s_r1, s_l1, s_r2, s_l2):
        s.wait_send()
    cp.wait()


def _ag_w(w_shard):
    k, n_per = w_shard.shape
    return pl.pallas_call(
        _ag_w_body,
        out_shape=jax.ShapeDtypeStruct((N_DEV, k, n_per), w_shard.dtype),
        in_specs=[pl.BlockSpec(memory_space=pl.ANY)],
        out_specs=pl.BlockSpec(memory_space=pl.ANY),
        scratch_shapes=[
            pltpu.SemaphoreType.DMA,
            pltpu.SemaphoreType.DMA((4,)),
            pltpu.SemaphoreType.DMA((4,)),
        ],
        compiler_params=pltpu.CompilerParams(collective_id=0),
    )(w_shard)


def _a2a_body(y_ref, out_ref, tcw_ref, tccw_ref, local_sem,
              send_sems, recv_sems):
    my = lax.axis_index("i")
    left = (my - 1) % N_DEV
    right = (my + 1) % N_DEV
    opp = (my + 2) % N_DEV
    _, m_per, n_per = y_ref.shape
    mh = m_per // 2
    top = pl.ds(0, mh)
    bot = pl.ds(mh, mh)

    def rows(r):
        return pl.ds(r * m_per, m_per)

    def rows_top(r):
        return pl.ds(r * m_per, mh)

    def rows_bot(r):
        return pl.ds(r * m_per + mh, mh)

    _neighbor_barrier(left, right)

    cp = pltpu.make_async_copy(y_ref.at[my], out_ref.at[rows(my), :],
                               local_sem)
    cp.start()

    sends = []
    s = _rdma(y_ref.at[right], out_ref.at[rows(my), :],
              send_sems.at[0], recv_sems.at[0], right)
    s.start()
    sends.append(s)
    s = _rdma(y_ref.at[left], out_ref.at[rows(my), :],
              send_sems.at[1], recv_sems.at[1], left)
    s.start()
    sends.append(s)
    s = _rdma(y_ref.at[opp, top], tcw_ref,
              send_sems.at[2], recv_sems.at[2], right)
    s.start()
    sends.append(s)
    s = _rdma(y_ref.at[opp, bot], tccw_ref,
              send_sems.at[3], recv_sems.at[3], left)
    s.start()
    sends.append(s)

    r = _rdma(tcw_ref, tcw_ref, send_sems.at[2], recv_sems.at[2], left)
    r.wait_recv()
    s = _rdma(tcw_ref, out_ref.at[rows_top(left), :],
              send_sems.at[4], recv_sems.at[4], right)
    s.start()
    sends.append(s)
    r = _rdma(tccw_ref, tccw_ref, send_sems.at[3], recv_sems.at[3], right)
    r.wait_recv()
    s = _rdma(tccw_ref, out_ref.at[rows_bot(right), :],
              send_sems.at[5], recv_sems.at[5], left)
    s.start()
    sends.append(s)

    r = _rdma(out_ref.at[rows(left), :], out_ref.at[rows(left), :],
              send_sems.at[0], recv_sems.at[0], left)
    r.wait_recv()
    r = _rdma(out_ref.at[rows(right), :], out_ref.at[rows(right), :],
              send_sems.at[1], recv_sems.at[1], right)
    r.wait_recv()
    r = _rdma(out_ref.at[rows_top(opp), :], out_ref.at[rows_top(opp), :],
              send_sems.at[4], recv_sems.at[4], left)
    r.wait_recv()
    r = _rdma(out_ref.at[rows_bot(opp), :], out_ref.at[rows_bot(opp), :],
              send_sems.at[5], recv_sems.at[5], right)
    r.wait_recv()

    for s in sends:
        s.wait_send()
    cp.wait()


def _a2a(y_blocks):
    _, m_per, n_per = y_blocks.shape
    return pl.pallas_call(
        _a2a_body,
        out_shape=jax.ShapeDtypeStruct((N_DEV * m_per, n_per),
                                       y_blocks.dtype),
        in_specs=[pl.BlockSpec(memory_space=pl.ANY)],
        out_specs=pl.BlockSpec(memory_space=pl.ANY),
        scratch_shapes=[
            pltpu.VMEM((m_per // 2, n_per), y_blocks.dtype),
            pltpu.VMEM((m_per // 2, n_per), y_blocks.dtype),
            pltpu.SemaphoreType.DMA,
            pltpu.SemaphoreType.DMA((6,)),
            pltpu.SemaphoreType.DMA((6,)),
        ],
        compiler_params=pltpu.CompilerParams(collective_id=1),
    )(y_blocks)


def _gelu(y):
    c = 0.7978845608028654
    return 0.5 * y * (1.0 + jnp.tanh(c * (y + 0.044715 * y * y * y)))


def kernel(x, w_mat):
    wg = _ag_w(w_mat)
    y = jnp.einsum("mk,ckn->cmn", x, wg,
                   preferred_element_type=jnp.float32)
    y = _gelu(y).astype(jnp.float32)
    return _a2a(y)


# device time: 1005997 ns/iter; 1.5443x vs baseline; 1.5443x over previous
import jax
import jax.numpy as jnp
from jax import lax
from jax.experimental import pallas as pl
from jax.experimental.pallas import tpu as pltpu

N_DEV = 4


def _neighbor_barrier(left, right):
    barrier_sem = pltpu.get_barrier_semaphore()
    for nbr in [left, right]:
        pl.semaphore_signal(
            barrier_sem, inc=1,
            device_id=(nbr,), device_id_type=pl.DeviceIdType.MESH,
        )
    pl.semaphore_wait(barrier_sem, 2)


def _rdma(src, dst, send_sem, recv_sem, dev):
    return pltpu.make_async_remote_copy(
        src_ref=src, dst_ref=dst, send_sem=send_sem, recv_sem=recv_sem,
        device_id=(dev,), device_id_type=pl.DeviceIdType.MESH,
    )


def _stage_copy(src, dst, vb_ref, sems, n_stages):
    rows = src.shape[0] // n_stages

    def rs(q):
        return pl.ds(q * rows, rows)

    for q in range(n_stages):
        slot = q % 2
        c1 = pltpu.make_async_copy(src.at[rs(q)], vb_ref.at[slot],
                                   sems.at[slot])
        c1.start()
        c1.wait()
        c2 = pltpu.make_async_copy(vb_ref.at[slot], dst.at[rs(q)],
                                   sems.at[2 + slot])
        c2.start()
        c2.wait()


def _ag_w_body(w_ref, wg_ref, vb_ref, local_sems, send_sems, recv_sems):
    my = lax.axis_index("i")
    left = (my - 1) % N_DEV
    right = (my + 1) % N_DEV
    opp = (my + 2) % N_DEV
    k, n_per = w_ref.shape
    half = k // 2
    top = pl.ds(0, half)
    bot = pl.ds(half, half)

    _neighbor_barrier(left, right)

    s_r1 = _rdma(w_ref, wg_ref.at[my], send_sems.at[0], recv_sems.at[0], right)
    s_l1 = _rdma(w_ref, wg_ref.at[my], send_sems.at[1], recv_sems.at[1], left)
    s_r1.start()
    s_l1.start()

    _stage_copy(w_ref, wg_ref.at[my], vb_ref, local_sems, 4)

    r_l1 = _rdma(wg_ref.at[left], wg_ref.at[left],
                 send_sems.at[0], recv_sems.at[0], left)
    r_l1.wait_recv()
    s_r2 = _rdma(wg_ref.at[left, top], wg_ref.at[left, top],
                 send_sems.at[2], recv_sems.at[2], right)
    s_r2.start()

    r_r1 = _rdma(wg_ref.at[right], wg_ref.at[right],
                 send_sems.at[1], recv_sems.at[1], right)
    r_r1.wait_recv()
    s_l2 = _rdma(wg_ref.at[right, bot], wg_ref.at[right, bot],
                 send_sems.at[3], recv_sems.at[3], left)
    s_l2.start()

    r_l2 = _rdma(wg_ref.at[opp, top], wg_ref.at[opp, top],
                 send_sems.at[2], recv_sems.at[2], left)
    r_l2.wait_recv()
    r_r2 = _rdma(wg_ref.at[opp, bot], wg_ref.at[opp, bot],
                 send_sems.at[3], recv_sems.at[3], right)
    r_r2.wait_recv()

    for s in (s_r1, s_l1, s_r2, s_l2):
        s.wait_send()


def _ag_w(w_shard):
    k, n_per = w_shard.shape
    return pl.pallas_call(
        _ag_w_body,
        out_shape=jax.ShapeDtypeStruct((N_DEV, k, n_per), w_shard.dtype),
        in_specs=[pl.BlockSpec(memory_space=pl.ANY)],
        out_specs=pl.BlockSpec(memory_space=pl.ANY),
        scratch_shapes=[
            pltpu.VMEM((2, k // 4, n_per), w_shard.dtype),
            pltpu.SemaphoreType.DMA((4,)),
            pltpu.SemaphoreType.DMA((4,)),
            pltpu.SemaphoreType.DMA((4,)),
        ],
        compiler_params=pltpu.CompilerParams(collective_id=0),
    )(w_shard)


def _a2a_body(y_ref, out_ref, tcw_ref, tccw_ref, vb_ref, local_sems,
              send_sems, recv_sems):
    my = lax.axis_index("i")
    left = (my - 1) % N_DEV
    right = (my + 1) % N_DEV
    opp = (my + 2) % N_DEV
    _, m_per, n_per = y_ref.shape
    mh = m_per // 2
    top = pl.ds(0, mh)
    bot = pl.ds(mh, mh)

    def rows(r):
        return pl.ds(r * m_per, m_per)

    def rows_top(r):
        return pl.ds(r * m_per, mh)

    def rows_bot(r):
        return pl.ds(r * m_per + mh, mh)

    _neighbor_barrier(left, right)

    sends = []
    s = _rdma(y_ref.at[right], out_ref.at[rows(my), :],
              send_sems.at[0], recv_sems.at[0], right)
    s.start()
    sends.append(s)
    s = _rdma(y_ref.at[left], out_ref.at[rows(my), :],
              send_sems.at[1], recv_sems.at[1], left)
    s.start()
    sends.append(s)
    s = _rdma(y_ref.at[opp, top], tcw_ref,
              send_sems.at[2], recv_sems.at[2], right)
    s.start()
    sends.append(s)
    s = _rdma(y_ref.at[opp, bot], tccw_ref,
              send_sems.at[3], recv_sems.at[3], left)
    s.start()
    sends.append(s)

    _stage_copy(y_ref.at[my], out_ref.at[rows(my), :], vb_ref,
                local_sems, 2)

    r = _rdma(tcw_ref, tcw_ref, send_sems.at[2], recv_sems.at[2], left)
    r.wait_recv()
    s = _rdma(tcw_ref, out_ref.at[rows_top(left), :],
              send_sems.at[4], recv_sems.at[4], right)
    s.start()
    sends.append(s)
    r = _rdma(tccw_ref, tccw_ref, send_sems.at[3], recv_sems.at[3], right)
    r.wait_recv()
    s = _rdma(tccw_ref, out_ref.at[rows_bot(right), :],
              send_sems.at[5], recv_sems.at[5], left)
    s.start()
    sends.append(s)

    r = _rdma(out_ref.at[rows(left), :], out_ref.at[rows(left), :],
              send_sems.at[0], recv_sems.at[0], left)
    r.wait_recv()
    r = _rdma(out_ref.at[rows(right), :], out_ref.at[rows(right), :],
              send_sems.at[1], recv_sems.at[1], right)
    r.wait_recv()
    r = _rdma(out_ref.at[rows_top(opp), :], out_ref.at[rows_top(opp), :],
              send_sems.at[4], recv_sems.at[4], left)
    r.wait_recv()
    r = _rdma(out_ref.at[rows_bot(opp), :], out_ref.at[rows_bot(opp), :],
              send_sems.at[5], recv_sems.at[5], right)
    r.wait_recv()

    for s in sends:
        s.wait_send()


def _a2a(y_blocks):
    _, m_per, n_per = y_blocks.shape
    return pl.pallas_call(
        _a2a_body,
        out_shape=jax.ShapeDtypeStruct((N_DEV * m_per, n_per),
                                       y_blocks.dtype),
        in_specs=[pl.BlockSpec(memory_space=pl.ANY)],
        out_specs=pl.BlockSpec(memory_space=pl.ANY),
        scratch_shapes=[
            pltpu.VMEM((m_per // 2, n_per), y_blocks.dtype),
            pltpu.VMEM((m_per // 2, n_per), y_blocks.dtype),
            pltpu.VMEM((2, m_per // 2, n_per), y_blocks.dtype),
            pltpu.SemaphoreType.DMA((4,)),
            pltpu.SemaphoreType.DMA((6,)),
            pltpu.SemaphoreType.DMA((6,)),
        ],
        compiler_params=pltpu.CompilerParams(collective_id=1),
    )(y_blocks)


def _gelu(y):
    c = 0.7978845608028654
    return 0.5 * y * (1.0 + jnp.tanh(c * (y + 0.044715 * y * y * y)))


def kernel(x, w_mat):
    wg = _ag_w(w_mat)
    y = jnp.einsum("mk,ckn->cmn", x, wg,
                   preferred_element_type=jnp.float32)
    y = _gelu(y).astype(jnp.float32)
    return _a2a(y)


# device time: 574077 ns/iter; 2.7062x vs baseline; 1.7524x over previous
import jax
import jax.numpy as jnp
from jax import lax
from jax.experimental import pallas as pl
from jax.experimental.pallas import tpu as pltpu

N_DEV = 4


def _neighbor_barrier(left, right):
    barrier_sem = pltpu.get_barrier_semaphore()
    for nbr in [left, right]:
        pl.semaphore_signal(
            barrier_sem, inc=1,
            device_id=(nbr,), device_id_type=pl.DeviceIdType.MESH,
        )
    pl.semaphore_wait(barrier_sem, 2)


def _rdma(src, dst, send_sem, recv_sem, dev):
    return pltpu.make_async_remote_copy(
        src_ref=src, dst_ref=dst, send_sem=send_sem, recv_sem=recv_sem,
        device_id=(dev,), device_id_type=pl.DeviceIdType.MESH,
    )


def _stage_copy(src, dst, vb_ref, sems, n_stages):
    rows = src.shape[0] // n_stages

    def rs(q):
        return pl.ds(q * rows, rows)

    for q in range(n_stages):
        slot = q % 2
        c1 = pltpu.make_async_copy(src.at[rs(q)], vb_ref.at[slot],
                                   sems.at[slot])
        c1.start()
        c1.wait()
        c2 = pltpu.make_async_copy(vb_ref.at[slot], dst.at[rs(q)],
                                   sems.at[2 + slot])
        c2.start()
        c2.wait()


def _ag_w_body(w_ref, wg_ref, vb_ref, local_sems, send_sems, recv_sems):
    my = lax.axis_index("i")
    left = (my - 1) % N_DEV
    right = (my + 1) % N_DEV
    opp = (my + 2) % N_DEV
    k, n_per = w_ref.shape
    half = k // 2
    top = pl.ds(0, half)
    bot = pl.ds(half, half)

    _neighbor_barrier(left, right)

    s_r1 = _rdma(w_ref, wg_ref.at[my], send_sems.at[0], recv_sems.at[0], right)
    s_l1 = _rdma(w_ref, wg_ref.at[my], send_sems.at[1], recv_sems.at[1], left)
    s_r1.start()
    s_l1.start()

    _stage_copy(w_ref, wg_ref.at[my], vb_ref, local_sems, 4)

    r_l1 = _rdma(wg_ref.at[left], wg_ref.at[left],
                 send_sems.at[0], recv_sems.at[0], left)
    r_l1.wait_recv()
    s_r2 = _rdma(wg_ref.at[left, top], wg_ref.at[left, top],
                 send_sems.at[2], recv_sems.at[2], right)
    s_r2.start()

    r_r1 = _rdma(wg_ref.at[right], wg_ref.at[right],
                 send_sems.at[1], recv_sems.at[1], right)
    r_r1.wait_recv()
    s_l2 = _rdma(wg_ref.at[right, bot], wg_ref.at[right, bot],
                 send_sems.at[3], recv_sems.at[3], left)
    s_l2.start()

    r_l2 = _rdma(wg_ref.at[opp, top], wg_ref.at[opp, top],
                 send_sems.at[2], recv_sems.at[2], left)
    r_l2.wait_recv()
    r_r2 = _rdma(wg_ref.at[opp, bot], wg_ref.at[opp, bot],
                 send_sems.at[3], recv_sems.at[3], right)
    r_r2.wait_recv()

    for s in (s_r1, s_l1, s_r2, s_l2):
        s.wait_send()


def _ag_w(w_shard):
    k, n_per = w_shard.shape
    return pl.pallas_call(
        _ag_w_body,
        out_shape=jax.ShapeDtypeStruct((N_DEV, k, n_per), w_shard.dtype),
        in_specs=[pl.BlockSpec(memory_space=pl.ANY)],
        out_specs=pl.BlockSpec(memory_space=pl.ANY),
        scratch_shapes=[
            pltpu.VMEM((2, k // 4, n_per), w_shard.dtype),
            pltpu.SemaphoreType.DMA((4,)),
            pltpu.SemaphoreType.DMA((4,)),
            pltpu.SemaphoreType.DMA((4,)),
        ],
        compiler_params=pltpu.CompilerParams(collective_id=0),
    )(w_shard)


def _a2a_body(y_ref, out_ref, tcw_ref, tccw_ref, vb_ref, local_sems,
              send_sems, recv_sems):
    my = lax.axis_index("i")
    left = (my - 1) % N_DEV
    right = (my + 1) % N_DEV
    opp = (my + 2) % N_DEV
    _, m_per, n_per = y_ref.shape
    mh = m_per // 2
    top = pl.ds(0, mh)
    bot = pl.ds(mh, mh)

    def rows(r):
        return pl.ds(r * m_per, m_per)

    def rows_top(r):
        return pl.ds(r * m_per, mh)

    def rows_bot(r):
        return pl.ds(r * m_per + mh, mh)

    _neighbor_barrier(left, right)

    sends = []
    s = _rdma(y_ref.at[right], out_ref.at[rows(my), :],
              send_sems.at[0], recv_sems.at[0], right)
    s.start()
    sends.append(s)
    s = _rdma(y_ref.at[left], out_ref.at[rows(my), :],
              send_sems.at[1], recv_sems.at[1], left)
    s.start()
    sends.append(s)
    s = _rdma(y_ref.at[opp, top], tcw_ref,
              send_sems.at[2], recv_sems.at[2], right)
    s.start()
    sends.append(s)
    s = _rdma(y_ref.at[opp, bot], tccw_ref,
              send_sems.at[3], recv_sems.at[3], left)
    s.start()
    sends.append(s)

    _stage_copy(y_ref.at[my], out_ref.at[rows(my), :], vb_ref,
                local_sems, 2)

    r = _rdma(tcw_ref, tcw_ref, send_sems.at[2], recv_sems.at[2], left)
    r.wait_recv()
    s = _rdma(tcw_ref, out_ref.at[rows_top(left), :],
              send_sems.at[4], recv_sems.at[4], right)
    s.start()
    sends.append(s)
    r = _rdma(tccw_ref, tccw_ref, send_sems.at[3], recv_sems.at[3], right)
    r.wait_recv()
    s = _rdma(tccw_ref, out_ref.at[rows_bot(right), :],
              send_sems.at[5], recv_sems.at[5], left)
    s.start()
    sends.append(s)

    r = _rdma(out_ref.at[rows(left), :], out_ref.at[rows(left), :],
              send_sems.at[0], recv_sems.at[0], left)
    r.wait_recv()
    r = _rdma(out_ref.at[rows(right), :], out_ref.at[rows(right), :],
              send_sems.at[1], recv_sems.at[1], right)
    r.wait_recv()
    r = _rdma(out_ref.at[rows_top(opp), :], out_ref.at[rows_top(opp), :],
              send_sems.at[4], recv_sems.at[4], left)
    r.wait_recv()
    r = _rdma(out_ref.at[rows_bot(opp), :], out_ref.at[rows_bot(opp), :],
              send_sems.at[5], recv_sems.at[5], right)
    r.wait_recv()

    for s in sends:
        s.wait_send()


def _a2a(y_blocks):
    _, m_per, n_per = y_blocks.shape
    return pl.pallas_call(
        _a2a_body,
        out_shape=jax.ShapeDtypeStruct((N_DEV * m_per, n_per),
                                       y_blocks.dtype),
        in_specs=[pl.BlockSpec(memory_space=pl.ANY)],
        out_specs=pl.BlockSpec(memory_space=pl.ANY),
        scratch_shapes=[
            pltpu.VMEM((m_per // 2, n_per), y_blocks.dtype),
            pltpu.VMEM((m_per // 2, n_per), y_blocks.dtype),
            pltpu.VMEM((2, m_per // 2, n_per), y_blocks.dtype),
            pltpu.SemaphoreType.DMA((4,)),
            pltpu.SemaphoreType.DMA((6,)),
            pltpu.SemaphoreType.DMA((6,)),
        ],
        compiler_params=pltpu.CompilerParams(collective_id=1),
    )(y_blocks)


def _gelu(y):
    c = 0.7978845608028654
    return 0.5 * y * (1.0 + jnp.tanh(c * (y + 0.044715 * y * y * y)))


def kernel(x, w_mat):
    wg = _ag_w(w_mat.astype(jnp.bfloat16))
    y = jnp.einsum("mk,ckn->cmn", x.astype(jnp.bfloat16), wg,
                   preferred_element_type=jnp.float32)
    y = _gelu(y).astype(jnp.bfloat16)
    return _a2a(y).astype(jnp.float32)
